# baseline (speedup 1.0000x reference)
import jax
import jax.numpy as jnp
from jax import lax
from jax.experimental import pallas as pl
from jax.experimental.pallas import tpu as pltpu
from jax.experimental.pallas import tpu_sc as plsc

VOCAB = 100000
D = 200
DP = 256
HW = 128
N_ROWS = 204800
N_HALF = N_ROWS * 2
NC, NS, L = 2, 16, 16
NW = NC * NS
PER_W = N_HALF // NW
CHUNK = 128
N_CHUNKS = PER_W // CHUNK
N_GROUPS = PER_W // L
CLAMP_EVEN = 2 * VOCAB


def _sc_body(idx_hbm, emb_hbm, out_hbm, idx_v, rows_v, sem):
    wid = lax.axis_index("s") * NC + lax.axis_index("c")
    base = wid * PER_W

    pltpu.sync_copy(idx_hbm.at[pl.ds(base, PER_W)], idx_v)

    clamp_v = jnp.full((L,), CLAMP_EVEN, jnp.int32)
    one_v = jnp.ones((L,), jnp.int32)

    def group_body(g, carry):
        v = idx_v[pl.ds(g * L, L)]
        idx_v[pl.ds(g * L, L)] = jnp.minimum(v, clamp_v + (v & one_v))
        return carry

    lax.fori_loop(0, N_GROUPS, group_body, jnp.int32(0))

    def chunk_body(c, carry):
        cbase = c * CHUNK
        pltpu.async_copy(
            emb_hbm.at[idx_v.at[pl.ds(cbase, CHUNK)]], rows_v, sem
        ).wait()
        pltpu.sync_copy(rows_v, out_hbm.at[pl.ds(base + cbase, CHUNK)])
        return carry

    lax.fori_loop(0, N_CHUNKS, chunk_body, jnp.int32(0))


@jax.jit
def _run(idx2, table_h):
    mesh = plsc.VectorSubcoreMesh(core_axis_name="c", subcore_axis_name="s")
    f = pl.kernel(
        _sc_body,
        out_type=jax.ShapeDtypeStruct((N_HALF, HW), jnp.float32),
        mesh=mesh,
        scratch_types=[
            pltpu.VMEM((PER_W,), jnp.int32),
            pltpu.VMEM((CHUNK, HW), jnp.float32),
            pltpu.SemaphoreType.DMA,
        ],
    )
    return f(idx2, table_h)


def kernel(word_indices, embedding_matrix, oov_embedding):
    idx = word_indices.reshape(-1).astype(jnp.int32)
    idx2 = (idx[:, None] * 2 + jnp.arange(2, dtype=jnp.int32)).reshape(-1)
    table_h = jnp.pad(
        jnp.concatenate([embedding_matrix, oov_embedding], axis=0),
        ((0, 0), (0, DP - D)),
    ).reshape(2 * (VOCAB + 1), HW)
    out = _run(idx2, table_h)
    return out.reshape(N_ROWS, DP)[:, :D].reshape(word_indices.shape + (D,))


# speedup vs baseline: 1.2453x; 1.2453x over previous
import jax
import jax.numpy as jnp
from jax import lax
from jax.experimental import pallas as pl
from jax.experimental.pallas import tpu as pltpu
from jax.experimental.pallas import tpu_sc as plsc

VOCAB = 100000
D = 200
DP = 256
N_ROWS = 204800
NC, NS, L = 2, 16, 16
NW = NC * NS
PER_W = N_ROWS // NW
CHUNK = 128
N_CHUNKS = PER_W // CHUNK
N_GROUPS = PER_W // L


def _sc_body(idx_hbm, emb_hbm, out_hbm, idx_v, rows_v, rows2_v, sem, sem2):
    wid = lax.axis_index("s") * NC + lax.axis_index("c")
    base = wid * PER_W

    pltpu.sync_copy(idx_hbm.at[pl.ds(base, PER_W)],
                    idx_v.at[pl.ds(0, PER_W)])

    vocab_v = jnp.full((L,), VOCAB, jnp.int32)
    zero_v = jnp.zeros((L,), jnp.int32)
    for k in range(CHUNK // L):
        idx_v[pl.ds(PER_W + k * L, L)] = zero_v

    def group_body(g, carry):
        v = idx_v[pl.ds(g * L, L)]
        idx_v[pl.ds(g * L, L)] = jnp.minimum(v, vocab_v)
        return carry

    lax.fori_loop(0, N_GROUPS, group_body, jnp.int32(0))

    def gstart(c, buf, sm):
        pltpu.async_copy(
            emb_hbm.at[idx_v.at[pl.ds(c * CHUNK, CHUNK)]], buf, sm)

    def gwait(buf, sm):
        pltpu.make_async_copy(
            emb_hbm.at[idx_v.at[pl.ds(0, CHUNK)]], buf, sm).wait()

    gstart(0, rows_v, sem)

    def chunk_body(c2, carry):
        c = 2 * c2
        gstart(c + 1, rows2_v, sem2)
        gwait(rows_v, sem)
        pltpu.sync_copy(rows_v, out_hbm.at[pl.ds(base + c * CHUNK, CHUNK)])
        gstart(c + 2, rows_v, sem)
        gwait(rows2_v, sem2)
        pltpu.sync_copy(rows2_v,
                        out_hbm.at[pl.ds(base + (c + 1) * CHUNK, CHUNK)])
        return carry

    lax.fori_loop(0, N_CHUNKS // 2, chunk_body, jnp.int32(0))
    gwait(rows_v, sem)


@jax.jit
def _run(idx_flat, table_p):
    mesh = plsc.VectorSubcoreMesh(core_axis_name="c", subcore_axis_name="s")
    f = pl.kernel(
        _sc_body,
        out_type=jax.ShapeDtypeStruct((N_ROWS, DP), jnp.float32),
        mesh=mesh,
        scratch_types=[
            pltpu.VMEM((PER_W + CHUNK,), jnp.int32),
            pltpu.VMEM((CHUNK, DP), jnp.float32),
            pltpu.VMEM((CHUNK, DP), jnp.float32),
            pltpu.SemaphoreType.DMA,
            pltpu.SemaphoreType.DMA,
        ],
    )
    return f(idx_flat, table_p)


def kernel(word_indices, embedding_matrix, oov_embedding):
    idx_flat = word_indices.reshape(-1).astype(jnp.int32)
    table_p = jnp.pad(
        jnp.concatenate([embedding_matrix, oov_embedding], axis=0),
        ((0, 0), (0, DP - D)),
    )
    out = _run(idx_flat, table_p)
    return out[:, :D].reshape(word_indices.shape + (D,))
